# baseline (speedup 1.0000x reference)
import jax
import jax.numpy as jnp
from jax.experimental import pallas as pl
from jax.experimental.pallas import tpu as pltpu

_B = 64
_D = 64
_M = 65536
_TILE = 16384
_T = _M // _TILE


def _body(x_ref, kt_ref, vt_ref, w_ref, p_ref, w_scr, sum_scr, acc_scr):
    p = pl.program_id(0)
    t = pl.program_id(1)

    @pl.when(jnp.logical_and(p == 0, t == 0))
    def _init():
        sum_scr[...] = jnp.zeros_like(sum_scr)
        acc_scr[...] = jnp.zeros_like(acc_scr)

    @pl.when(p == 0)
    def _compute():
        xv = x_ref[...]
        ktb = kt_ref[...].astype(jnp.bfloat16)
        xb = xv.astype(jnp.bfloat16)
        num = jax.lax.dot_general(
            xb, ktb, (((1,), (0,)), ((), ())),
            preferred_element_type=jnp.float32)
        c = 10.0 * 1.4426950408889634
        inv_xn = c * jax.lax.rsqrt(
            jnp.sum(xv * xv, axis=1, keepdims=True))
        k2 = jax.lax.dot_general(
            jnp.ones((1, _D), jnp.bfloat16), ktb * ktb,
            (((1,), (0,)), ((), ())),
            preferred_element_type=jnp.float32)
        inv_kn = jax.lax.rsqrt(k2)
        scale = jnp.minimum(inv_xn * inv_kn, c * 1e8)
        e = jnp.exp2(num * scale)
        eb = e.astype(jnp.bfloat16)
        w_scr[t] = eb
        sum_scr[...] += jax.lax.dot_general(
            eb, jnp.ones((_TILE, 1), jnp.bfloat16), (((1,), (0,)), ((), ())),
            preferred_element_type=jnp.float32)

    @pl.when(p == 1)
    def _normalize():
        eb = w_scr[t]
        vtb = vt_ref[...].astype(jnp.bfloat16)
        acc_scr[...] += jax.lax.dot_general(
            eb, vtb, (((1,), (1,)), ((), ())),
            preferred_element_type=jnp.float32)
        inv = 1.0 / sum_scr[...]
        w_ref[...] = eb.astype(jnp.float32) * inv
        p_ref[...] = acc_scr[...] * inv


@jax.jit
def kernel(x, memory_keys, memory_values):
    kt = memory_keys.T
    vt = memory_values.T
    weights, pred = pl.pallas_call(
        _body,
        grid=(2, _T),
        in_specs=[
            pl.BlockSpec((_B, _D), lambda p, t: (0, 0)),
            pl.BlockSpec((_D, _TILE), lambda p, t: (0, t * (1 - p))),
            pl.BlockSpec((_D, _TILE), lambda p, t: (0, t * p)),
        ],
        out_specs=[
            pl.BlockSpec((_B, _TILE), lambda p, t: (0, t * p)),
            pl.BlockSpec((_B, _D), lambda p, t: (0, 0)),
        ],
        out_shape=[
            jax.ShapeDtypeStruct((_B, _M), jnp.float32),
            jax.ShapeDtypeStruct((_B, _D), jnp.float32),
        ],
        scratch_shapes=[
            pltpu.VMEM((_T, _B, _TILE), jnp.bfloat16),
            pltpu.VMEM((_B, 1), jnp.float32),
            pltpu.VMEM((_B, _D), jnp.float32),
        ],
    )(x, kt, vt)
    return (pred, weights)


# speedup vs baseline: 1.0109x; 1.0109x over previous
import jax
import jax.numpy as jnp
from jax.experimental import pallas as pl
from jax.experimental.pallas import tpu as pltpu

_B = 64
_D = 64
_M = 65536
_TILE = 16384
_T = _M // _TILE


def _body(x_ref, kt_ref, vt_ref, w_ref, p_ref, w_scr, sum_scr, acc_scr):
    p = pl.program_id(0)
    t = pl.program_id(1)

    @pl.when(jnp.logical_and(p == 0, t == 0))
    def _init():
        sum_scr[...] = jnp.zeros_like(sum_scr)
        acc_scr[...] = jnp.zeros_like(acc_scr)

    @pl.when(p == 0)
    def _compute():
        xv = x_ref[...]
        ktb = kt_ref[...].astype(jnp.bfloat16)
        vtb = vt_ref[...].astype(jnp.bfloat16)
        xb = xv.astype(jnp.bfloat16)
        num = jax.lax.dot_general(
            xb, ktb, (((1,), (0,)), ((), ())),
            preferred_element_type=jnp.float32)
        c = 10.0 * 1.4426950408889634
        inv_xn = c * jax.lax.rsqrt(
            jnp.sum(xv * xv, axis=1, keepdims=True))
        k2 = jax.lax.dot_general(
            jnp.ones((1, _D), jnp.bfloat16), ktb * ktb,
            (((1,), (0,)), ((), ())),
            preferred_element_type=jnp.float32)
        inv_kn = jax.lax.rsqrt(k2)
        scale = jnp.minimum(inv_xn * inv_kn, c * 1e8)
        e = jnp.exp2(num * scale)
        eb = e.astype(jnp.bfloat16)
        w_scr[t] = eb
        sum_scr[...] += jax.lax.dot_general(
            eb, jnp.ones((_TILE, 1), jnp.bfloat16), (((1,), (0,)), ((), ())),
            preferred_element_type=jnp.float32)
        acc_scr[...] += jax.lax.dot_general(
            eb, vtb, (((1,), (1,)), ((), ())),
            preferred_element_type=jnp.float32)

    @pl.when(p == 1)
    def _normalize():
        inv = 1.0 / sum_scr[...]
        w_ref[...] = w_scr[t].astype(jnp.float32) * inv
        p_ref[...] = acc_scr[...] * inv


@jax.jit
def kernel(x, memory_keys, memory_values):
    kt = memory_keys.T
    vt = memory_values.T
    weights, pred = pl.pallas_call(
        _body,
        grid=(2, _T),
        in_specs=[
            pl.BlockSpec((_B, _D), lambda p, t: (0, 0)),
            pl.BlockSpec((_D, _TILE), lambda p, t: (0, t * (1 - p))),
            pl.BlockSpec((_D, _TILE), lambda p, t: (0, t * (1 - p))),
        ],
        out_specs=[
            pl.BlockSpec((_B, _TILE), lambda p, t: (0, t * p)),
            pl.BlockSpec((_B, _D), lambda p, t: (0, 0)),
        ],
        out_shape=[
            jax.ShapeDtypeStruct((_B, _M), jnp.float32),
            jax.ShapeDtypeStruct((_B, _D), jnp.float32),
        ],
        scratch_shapes=[
            pltpu.VMEM((_T, _B, _TILE), jnp.bfloat16),
            pltpu.VMEM((_B, 1), jnp.float32),
            pltpu.VMEM((_B, _D), jnp.float32),
        ],
    )(x, kt, vt)
    return (pred, weights)
